# baseline (speedup 1.0000x reference)
import functools

import jax
import jax.numpy as jnp
from jax import lax
from jax.experimental import pallas as pl
from jax.experimental.pallas import tpu as pltpu
from jax.experimental.pallas import tpu_sc as plsc

N_SITES = 200
D_MODEL = 64
BATCH = 4096

NC, NS, L = 2, 16, 16
NW = NC * NS
TOK_TOTAL = BATCH * N_SITES
TOK_PER_W = TOK_TOTAL // NW
CHUNK = 1600
N_CHUNKS = TOK_PER_W // CHUNK
GATHER_W = 128
N_GATHERS = -(-CHUNK // GATHER_W)
CHUNK_PAD = N_GATHERS * GATHER_W
N_GROUPS = CHUNK // L


def _sc_embed(down_flat, up_flat, table):
    mesh = plsc.VectorSubcoreMesh(
        core_axis_name="c", subcore_axis_name="s", num_cores=NC, num_subcores=NS
    )

    @functools.partial(
        pl.kernel,
        out_type=jax.ShapeDtypeStruct((TOK_TOTAL, D_MODEL), jnp.float32),
        mesh=mesh,
        scratch_types=[
            pltpu.VMEM((CHUNK,), jnp.int32),
            pltpu.VMEM((CHUNK,), jnp.int32),
            pltpu.VMEM((CHUNK_PAD,), jnp.int32),
            pltpu.VMEM((CHUNK_PAD, D_MODEL), jnp.float32),
            pltpu.SemaphoreType.DMA,
        ],
        compiler_params=pltpu.CompilerParams(use_tc_tiling_on_sc=False),
    )
    def k(down_hbm, up_hbm, table_hbm, out_hbm, d_v, u_v, idx_v, rows_v, sem):
        wid = lax.axis_index("s") * NC + lax.axis_index("c")
        base = wid * TOK_PER_W

        zeros = jnp.zeros((L,), jnp.int32)
        for g in range((CHUNK_PAD - CHUNK) // L):
            idx_v[pl.ds(CHUNK + g * L, L)] = zeros

        def chunk_body(ci, _):
            t0 = base + ci * CHUNK
            pltpu.sync_copy(down_hbm.at[pl.ds(t0, CHUNK)], d_v)
            pltpu.sync_copy(up_hbm.at[pl.ds(t0, CHUNK)], u_v)

            def grp_body(g, _):
                o = pl.multiple_of(g * L, L)
                d = d_v[pl.ds(o, L)]
                u = u_v[pl.ds(o, L)]
                idx_v[pl.ds(o, L)] = u + d + d
                return 0

            lax.fori_loop(0, N_GROUPS, grp_body, 0)

            descs = []
            for j in range(N_GATHERS):
                descs.append(
                    pltpu.async_copy(
                        table_hbm.at[idx_v.at[pl.ds(j * GATHER_W, GATHER_W)]],
                        rows_v.at[pl.ds(j * GATHER_W, GATHER_W)],
                        sem,
                    )
                )
            for desc in descs:
                desc.wait()

            pltpu.sync_copy(rows_v.at[pl.ds(0, CHUNK)], out_hbm.at[pl.ds(t0, CHUNK)])
            return 0

        lax.fori_loop(0, N_CHUNKS, chunk_body, 0)

    return k(down_flat, up_flat, table)


def kernel(n_flat, embed_table):
    n = jnp.asarray(n_flat)
    down = n[:, :N_SITES].reshape(TOK_TOTAL)
    up = n[:, N_SITES:].reshape(TOK_TOTAL)
    out = _sc_embed(down, up, embed_table)
    return out.reshape(BATCH, N_SITES, D_MODEL)


# speedup vs baseline: 13.2545x; 13.2545x over previous
import functools

import jax
import jax.numpy as jnp
from jax import lax
from jax.experimental import pallas as pl
from jax.experimental.pallas import tpu as pltpu
from jax.experimental.pallas import tpu_sc as plsc

N_SITES = 200
D_MODEL = 64
BATCH = 4096

NC, NS, L = 2, 16, 16
NW = NC * NS
TOK_TOTAL = BATCH * N_SITES
TOK_PER_W = TOK_TOTAL // NW
CHUNK = 1600
N_CHUNKS = TOK_PER_W // CHUNK
GATHER_W = 128
N_GATHERS = -(-CHUNK // GATHER_W)
CHUNK_PAD = N_GATHERS * GATHER_W
N_GROUPS = CHUNK // L


def _sc_embed(down_flat, up_flat, table):
    mesh = plsc.VectorSubcoreMesh(
        core_axis_name="c", subcore_axis_name="s", num_cores=NC, num_subcores=NS
    )

    @functools.partial(
        pl.kernel,
        out_type=jax.ShapeDtypeStruct((TOK_TOTAL, D_MODEL), jnp.float32),
        mesh=mesh,
        scratch_types=[
            pltpu.VMEM((CHUNK,), jnp.int32),
            pltpu.VMEM((CHUNK,), jnp.int32),
            pltpu.VMEM((CHUNK_PAD,), jnp.int32),
            pltpu.VMEM((CHUNK_PAD, D_MODEL), jnp.float32),
            pltpu.VMEM_SHARED((4, D_MODEL), jnp.float32),
            pltpu.SemaphoreType.DMA,
        ],
        compiler_params=pltpu.CompilerParams(use_tc_tiling_on_sc=False),
    )
    def k(down_hbm, up_hbm, table_hbm, out_hbm, d_v, u_v, idx_v, rows_v, tab_v, sem):
        wid = lax.axis_index("s") * NC + lax.axis_index("c")
        base = wid * TOK_PER_W

        @pl.when(lax.axis_index("s") == 0)
        def _stage():
            pltpu.sync_copy(table_hbm, tab_v)

        plsc.subcore_barrier()

        zeros = jnp.zeros((L,), jnp.int32)
        for g in range((CHUNK_PAD - CHUNK) // L):
            idx_v[pl.ds(CHUNK + g * L, L)] = zeros

        def chunk_body(ci, _):
            t0 = base + ci * CHUNK
            pltpu.sync_copy(down_hbm.at[pl.ds(t0, CHUNK)], d_v)
            pltpu.sync_copy(up_hbm.at[pl.ds(t0, CHUNK)], u_v)

            def grp_body(g, _):
                o = pl.multiple_of(g * L, L)
                d = d_v[pl.ds(o, L)]
                u = u_v[pl.ds(o, L)]
                idx_v[pl.ds(o, L)] = u + d + d
                return 0

            lax.fori_loop(0, N_GROUPS, grp_body, 0)

            descs = []
            for j in range(N_GATHERS):
                descs.append(
                    pltpu.async_copy(
                        tab_v.at[idx_v.at[pl.ds(j * GATHER_W, GATHER_W)]],
                        rows_v.at[pl.ds(j * GATHER_W, GATHER_W)],
                        sem,
                    )
                )
            for desc in descs:
                desc.wait()

            pltpu.sync_copy(rows_v.at[pl.ds(0, CHUNK)], out_hbm.at[pl.ds(t0, CHUNK)])
            return 0

        lax.fori_loop(0, N_CHUNKS, chunk_body, 0)

    return k(down_flat, up_flat, table)


def kernel(n_flat, embed_table):
    n = jnp.asarray(n_flat)
    down = n[:, :N_SITES].reshape(TOK_TOTAL)
    up = n[:, N_SITES:].reshape(TOK_TOTAL)
    out = _sc_embed(down, up, embed_table)
    return out.reshape(BATCH, N_SITES, D_MODEL)


# speedup vs baseline: 13.7672x; 1.0387x over previous
import functools

import jax
import jax.numpy as jnp
from jax import lax
from jax.experimental import pallas as pl
from jax.experimental.pallas import tpu as pltpu
from jax.experimental.pallas import tpu_sc as plsc

N_SITES = 200
D_MODEL = 64
BATCH = 4096

NC, NS, L = 2, 16, 16
NW = NC * NS
TOK_TOTAL = BATCH * N_SITES
TOK_PER_W = TOK_TOTAL // NW
CHUNK = 640
N_CHUNKS = TOK_PER_W // CHUNK
GATHER_W = 128
N_GATHERS = CHUNK // GATHER_W
N_GROUPS = CHUNK // L
NBUF = 2


def _sc_embed(down_flat, up_flat, table):
    mesh = plsc.VectorSubcoreMesh(
        core_axis_name="c", subcore_axis_name="s", num_cores=NC, num_subcores=NS
    )

    scratch = []
    for _ in range(NBUF):
        scratch += [
            pltpu.VMEM((CHUNK,), jnp.int32),
            pltpu.VMEM((CHUNK,), jnp.int32),
            pltpu.VMEM((CHUNK,), jnp.int32),
            pltpu.VMEM((CHUNK, D_MODEL), jnp.float32),
            pltpu.SemaphoreType.DMA,
            pltpu.SemaphoreType.DMA,
        ]
    scratch.append(pltpu.VMEM_SHARED((4, D_MODEL), jnp.float32))

    @functools.partial(
        pl.kernel,
        out_type=jax.ShapeDtypeStruct((TOK_TOTAL, D_MODEL), jnp.float32),
        mesh=mesh,
        scratch_types=scratch,
        compiler_params=pltpu.CompilerParams(use_tc_tiling_on_sc=False),
    )
    def k(down_hbm, up_hbm, table_hbm, out_hbm, *s):
        bufs = [s[i * 6:(i + 1) * 6] for i in range(NBUF)]
        tab_v = s[NBUF * 6]

        wid = lax.axis_index("s") * NC + lax.axis_index("c")
        base = wid * TOK_PER_W

        @pl.when(lax.axis_index("s") == 0)
        def _stage():
            pltpu.sync_copy(table_hbm, tab_v)

        plsc.subcore_barrier()

        def pair_body(cp, _):
            for b in range(NBUF):
                d_v, u_v, idx_v, rows_v, sem_g, sem_o = bufs[b]
                c = cp * NBUF + b
                t0 = base + c * CHUNK

                pltpu.sync_copy(down_hbm.at[pl.ds(t0, CHUNK)], d_v)
                pltpu.sync_copy(up_hbm.at[pl.ds(t0, CHUNK)], u_v)

                def grp_body(g, _):
                    o = pl.multiple_of(g * L, L)
                    d = d_v[pl.ds(o, L)]
                    u = u_v[pl.ds(o, L)]
                    idx_v[pl.ds(o, L)] = u + d + d
                    return 0

                lax.fori_loop(0, N_GROUPS, grp_body, 0)

                @pl.when(cp > 0)
                def _drain_prev():
                    pltpu.make_async_copy(
                        rows_v, out_hbm.at[pl.ds(0, CHUNK)], sem_o
                    ).wait()

                descs = []
                for j in range(N_GATHERS):
                    descs.append(
                        pltpu.async_copy(
                            tab_v.at[idx_v.at[pl.ds(j * GATHER_W, GATHER_W)]],
                            rows_v.at[pl.ds(j * GATHER_W, GATHER_W)],
                            sem_g,
                        )
                    )
                for desc in descs:
                    desc.wait()

                pltpu.async_copy(rows_v, out_hbm.at[pl.ds(t0, CHUNK)], sem_o)
            return 0

        lax.fori_loop(0, N_CHUNKS // NBUF, pair_body, 0)

        for b in range(NBUF):
            _, _, _, rows_v, _, sem_o = bufs[b]
            pltpu.make_async_copy(
                rows_v, out_hbm.at[pl.ds(0, CHUNK)], sem_o
            ).wait()

    return k(down_flat, up_flat, table)


def kernel(n_flat, embed_table):
    n = jnp.asarray(n_flat)
    down = n[:, :N_SITES].reshape(TOK_TOTAL)
    up = n[:, N_SITES:].reshape(TOK_TOTAL)
    out = _sc_embed(down, up, embed_table)
    return out.reshape(BATCH, N_SITES, D_MODEL)
